# baseline (speedup 1.0000x reference)
import functools

import jax
import jax.numpy as jnp
from jax import lax
from jax.experimental import pallas as pl
from jax.experimental.pallas import tpu as pltpu
from jax.experimental.pallas import tpu_sc as plsc

N = 10000
E = 320000
NC = 2
NS = 16
NW = NC * NS
EPW = E // NW
ROWS_PW = 624
NBUF = 5
K16 = 125

_sc_mesh = plsc.VectorSubcoreMesh(core_axis_name="c", subcore_axis_name="s",
                                  num_cores=NC, num_subcores=NS)


def _make_prop(D, K, tc_tiling):
    NCHUNK = EPW // K
    NGROUP = NCHUNK // NBUF
    assert NCHUNK * K == EPW and NGROUP * NBUF == NCHUNK and K <= 128

    @functools.partial(
        pl.kernel,
        out_type=jax.ShapeDtypeStruct((NC, N, D), jnp.float32),
        mesh=_sc_mesh,
        scratch_types=[
            pltpu.VMEM_SHARED((N, D), jnp.float32),
            pltpu.VMEM((NCHUNK, K), jnp.int32),
            pltpu.VMEM((NCHUNK, K), jnp.int32),
            pltpu.VMEM((NBUF, K, D), jnp.float32),
            pltpu.SemaphoreType.DMA((NBUF,)),
            pltpu.SemaphoreType.DMA((NBUF,)),
        ],
        compiler_params=pltpu.CompilerParams(use_tc_tiling_on_sc=tc_tiling),
    )
    def _prop(h_hbm, src_hbm, dst_hbm, out_hbm, acc, sidx, didx, rows,
              gsem, ssem):
        c = lax.axis_index("c")
        s = lax.axis_index("s")
        wid = c * NS + s

        pltpu.sync_copy(src_hbm.at[wid], sidx)
        pltpu.sync_copy(dst_hbm.at[wid], didx)

        r0 = pl.multiple_of(s * ROWS_PW, 8)
        pltpu.sync_copy(h_hbm.at[pl.ds(r0, ROWS_PW)], acc.at[pl.ds(r0, ROWS_PW)])

        @pl.when(s == 0)
        def _init_tail():
            t0 = NS * ROWS_PW
            pltpu.sync_copy(h_hbm.at[pl.ds(t0, N - t0)],
                            acc.at[pl.ds(t0, N - t0)])

        plsc.subcore_barrier()

        for b in range(NBUF):
            pltpu.async_copy(h_hbm.at[sidx.at[b]], rows.at[b], gsem.at[b])

        def _group(g, carry):
            j0 = g * NBUF
            descs = []
            for b in range(NBUF):
                pltpu.make_async_copy(h_hbm.at[sidx.at[b]], rows.at[b],
                                      gsem.at[b]).wait()
                descs.append(pltpu.async_copy(
                    rows.at[b], acc.at[didx.at[j0 + b]], ssem.at[b],
                    add=True))
            for b in range(NBUF):
                descs[b].wait()

                @pl.when(g < NGROUP - 1)
                def _next_gather(b=b):
                    pltpu.async_copy(h_hbm.at[sidx.at[j0 + NBUF + b]],
                                     rows.at[b], gsem.at[b])

            return carry

        lax.fori_loop(0, NGROUP, _group, 0)
        plsc.subcore_barrier()

        pltpu.sync_copy(acc.at[pl.ds(r0, ROWS_PW)],
                        out_hbm.at[c, pl.ds(r0, ROWS_PW)])

        @pl.when(s == 0)
        def _out_tail():
            t0 = NS * ROWS_PW
            pltpu.sync_copy(acc.at[pl.ds(t0, N - t0)],
                            out_hbm.at[c, pl.ds(t0, N - t0)])

    return _prop


_prop16 = _make_prop(16, K16, False)


K64 = 125
NCHUNK64 = EPW // K64
NGROUP64 = NCHUNK64 // NBUF


@functools.partial(
    pl.kernel,
    out_type=jax.ShapeDtypeStruct((NC, 2, N, 64), jnp.float32),
    mesh=_sc_mesh,
    scratch_types=[
        pltpu.VMEM_SHARED((N, 64), jnp.float32),
        pltpu.VMEM((NCHUNK64, K64), jnp.int32),
        pltpu.VMEM((NCHUNK64, K64), jnp.int32),
        pltpu.VMEM((NBUF, K64, 64), jnp.float32),
        pltpu.SemaphoreType.DMA((NBUF,)),
        pltpu.SemaphoreType.DMA((NBUF,)),
    ],
    compiler_params=pltpu.CompilerParams(use_tc_tiling_on_sc=False),
)
def _prop64x2(h_hbm, src_hbm, dst_hbm, out_hbm, acc, sidx, didx, rows,
              gsem, ssem):
    c = lax.axis_index("c")
    s = lax.axis_index("s")
    wid = c * NS + s

    pltpu.sync_copy(src_hbm.at[wid], sidx)
    pltpu.sync_copy(dst_hbm.at[wid], didx)

    r0 = pl.multiple_of(s * ROWS_PW, 8)
    t0 = NS * ROWS_PW

    for h in range(2):
        tab = h_hbm.at[h]
        pltpu.sync_copy(tab.at[pl.ds(r0, ROWS_PW)], acc.at[pl.ds(r0, ROWS_PW)])

        @pl.when(s == 0)
        def _init_tail(tab=tab):
            pltpu.sync_copy(tab.at[pl.ds(t0, N - t0)], acc.at[pl.ds(t0, N - t0)])

        plsc.subcore_barrier()

        for b in range(NBUF):
            pltpu.async_copy(tab.at[sidx.at[b]], rows.at[b], gsem.at[b])

        def _group(g, carry, tab=tab):
            j0 = g * NBUF
            descs = []
            for b in range(NBUF):
                pltpu.make_async_copy(tab.at[sidx.at[b]], rows.at[b],
                                      gsem.at[b]).wait()
                descs.append(pltpu.async_copy(
                    rows.at[b], acc.at[didx.at[j0 + b]], ssem.at[b],
                    add=True))
            for b in range(NBUF):
                descs[b].wait()

                @pl.when(g < NGROUP64 - 1)
                def _next_gather(b=b, tab=tab):
                    pltpu.async_copy(tab.at[sidx.at[j0 + NBUF + b]],
                                     rows.at[b], gsem.at[b])

            return carry

        lax.fori_loop(0, NGROUP64, _group, 0)
        plsc.subcore_barrier()

        pltpu.sync_copy(acc.at[pl.ds(r0, ROWS_PW)],
                        out_hbm.at[c, h, pl.ds(r0, ROWS_PW)])

        @pl.when(s == 0)
        def _out_tail(h=h):
            pltpu.sync_copy(acc.at[pl.ds(t0, N - t0)],
                            out_hbm.at[c, h, pl.ds(t0, N - t0)])

        plsc.subcore_barrier()


KD = 125
NCHUNK_D = EPW // KD
NGROUP_D = NCHUNK_D // NBUF


@functools.partial(
    pl.kernel,
    out_type=jax.ShapeDtypeStruct((NC, N, 8), jnp.float32),
    mesh=_sc_mesh,
    scratch_types=[
        pltpu.VMEM_SHARED((N, 8), jnp.float32),
        pltpu.VMEM((NCHUNK_D, KD), jnp.int32),
        pltpu.VMEM((ROWS_PW, 8), jnp.float32),
        pltpu.SemaphoreType.DMA((NBUF,)),
    ],
    compiler_params=pltpu.CompilerParams(use_tc_tiling_on_sc=False),
)
def _deg_kernel(dst_hbm, ones_hbm, out_hbm, acc, didx, ones_v, ssem):
    c = lax.axis_index("c")
    s = lax.axis_index("s")
    wid = c * NS + s

    pltpu.sync_copy(dst_hbm.at[wid], didx)
    pltpu.sync_copy(ones_hbm, ones_v)

    r0 = pl.multiple_of(s * ROWS_PW, 8)
    pltpu.sync_copy(ones_v, acc.at[pl.ds(r0, ROWS_PW)])

    @pl.when(s == 0)
    def _init_tail():
        t0 = NS * ROWS_PW
        pltpu.sync_copy(ones_v.at[pl.ds(0, N - t0)], acc.at[pl.ds(t0, N - t0)])

    plsc.subcore_barrier()

    src_rows = ones_v.at[pl.ds(0, KD)]

    def _group(g, carry):
        j0 = g * NBUF
        descs = []
        for b in range(NBUF):
            descs.append(pltpu.async_copy(
                src_rows, acc.at[didx.at[j0 + b]], ssem.at[b], add=True))
        for b in range(NBUF):
            descs[b].wait()
        return carry

    lax.fori_loop(0, NGROUP_D, _group, 0)
    plsc.subcore_barrier()

    pltpu.sync_copy(acc.at[pl.ds(r0, ROWS_PW)], out_hbm.at[c, pl.ds(r0, ROWS_PW)])

    @pl.when(s == 0)
    def _out_tail():
        t0 = NS * ROWS_PW
        pltpu.sync_copy(acc.at[pl.ds(t0, N - t0)], out_hbm.at[c, pl.ds(t0, N - t0)])


def _tc1_body(degq_ref, x_ref, w1_ref, h1s_ref, dinv_ref):
    deg = degq_ref[0, :, 0] + degq_ref[1, :, 0] - 1.0
    dinv = lax.rsqrt(deg)
    dinv_ref[...] = dinv
    h1 = jnp.dot(x_ref[...], w1_ref[...], preferred_element_type=jnp.float32)
    h1s = h1 * dinv[:, None]
    h1s_ref[0] = h1s[:, :64]
    h1s_ref[1] = h1s[:, 64:]


def _tc2_body(p_ref, h1s_ref, dinv_ref, b1_ref, w2_ref, h2s_ref):
    dinv = dinv_ref[...]
    raw2 = p_ref[0] + p_ref[1] - h1s_ref[...]
    raw = jnp.concatenate([raw2[0], raw2[1]], axis=1)
    out1 = raw * dinv[:, None] + b1_ref[...][None, :]
    a = jnp.maximum(out1, 0.0)
    h2 = jnp.dot(a, w2_ref[...], preferred_element_type=jnp.float32)
    h2s_ref[...] = h2 * dinv[:, None]


def _tc3_body(q_ref, h2s_ref, dinv_ref, b2_ref, out_ref):
    raw = q_ref[0] + q_ref[1] - h2s_ref[...]
    out_ref[...] = raw * dinv_ref[...][:, None] + b2_ref[...][None, :]


_tc1 = pl.pallas_call(
    _tc1_body,
    out_shape=(jax.ShapeDtypeStruct((2, N, 64), jnp.float32),
               jax.ShapeDtypeStruct((N,), jnp.float32)),
)

_tc2 = pl.pallas_call(
    _tc2_body,
    out_shape=jax.ShapeDtypeStruct((N, 16), jnp.float32),
)

_tc3 = pl.pallas_call(
    _tc3_body,
    out_shape=jax.ShapeDtypeStruct((N, 16), jnp.float32),
)


def kernel(x, train_pos_edge_index, W1, b1, W2, b2):
    src = train_pos_edge_index[0].astype(jnp.int32)
    dst = train_pos_edge_index[1].astype(jnp.int32)
    src16 = src.reshape(NW, EPW // K16, K16)
    dst16 = dst.reshape(NW, EPW // K16, K16)
    ones = jnp.ones((ROWS_PW, 8), jnp.float32)
    degq = _deg_kernel(dst16, ones)
    h1s, dinv = _tc1(degq, x, W1)
    p = _prop64x2(h1s, src16, dst16)
    h2s = _tc2(p, h1s, dinv, b1, W2)
    q = _prop16(h2s, src16, dst16)
    return _tc3(q, h2s, dinv, b2)


# speedup vs baseline: 1.1589x; 1.1589x over previous
import functools

import jax
import jax.numpy as jnp
from jax import lax
from jax.experimental import pallas as pl
from jax.experimental.pallas import tpu as pltpu
from jax.experimental.pallas import tpu_sc as plsc

N = 10000
E = 320000
NC = 2
NS = 16
NW = NC * NS
EPW = E // NW
ROWS_PW = 624
NBUF = 5
K128 = 40
K16 = 125

_sc_mesh = plsc.VectorSubcoreMesh(core_axis_name="c", subcore_axis_name="s",
                                  num_cores=NC, num_subcores=NS)


def _make_prop(D, K, tc_tiling, NBUF=NBUF):
    NCHUNK = EPW // K
    NGROUP = NCHUNK // NBUF
    assert NCHUNK * K == EPW and NGROUP * NBUF == NCHUNK and K <= 128

    @functools.partial(
        pl.kernel,
        out_type=jax.ShapeDtypeStruct((NC, N, D), jnp.float32),
        mesh=_sc_mesh,
        scratch_types=[
            pltpu.VMEM_SHARED((N, D), jnp.float32),
            pltpu.VMEM((NCHUNK, K), jnp.int32),
            pltpu.VMEM((NCHUNK, K), jnp.int32),
            pltpu.VMEM((NBUF, K, D), jnp.float32),
            pltpu.SemaphoreType.DMA((NBUF,)),
            pltpu.SemaphoreType.DMA((NBUF,)),
        ],
        compiler_params=pltpu.CompilerParams(use_tc_tiling_on_sc=tc_tiling),
    )
    def _prop(h_hbm, src_hbm, dst_hbm, out_hbm, acc, sidx, didx, rows,
              gsem, ssem):
        c = lax.axis_index("c")
        s = lax.axis_index("s")
        wid = c * NS + s

        pltpu.sync_copy(src_hbm.at[wid], sidx)
        pltpu.sync_copy(dst_hbm.at[wid], didx)

        r0 = pl.multiple_of(s * ROWS_PW, 8)
        pltpu.sync_copy(h_hbm.at[pl.ds(r0, ROWS_PW)], acc.at[pl.ds(r0, ROWS_PW)])

        @pl.when(s == 0)
        def _init_tail():
            t0 = NS * ROWS_PW
            pltpu.sync_copy(h_hbm.at[pl.ds(t0, N - t0)],
                            acc.at[pl.ds(t0, N - t0)])

        plsc.subcore_barrier()

        for b in range(NBUF):
            pltpu.async_copy(h_hbm.at[sidx.at[b]], rows.at[b], gsem.at[b])

        def _group(g, carry):
            j0 = g * NBUF
            descs = []
            for b in range(NBUF):
                pltpu.make_async_copy(h_hbm.at[sidx.at[b]], rows.at[b],
                                      gsem.at[b]).wait()
                descs.append(pltpu.async_copy(
                    rows.at[b], acc.at[didx.at[j0 + b]], ssem.at[b],
                    add=True))
            for b in range(NBUF):
                descs[b].wait()

                @pl.when(g < NGROUP - 1)
                def _next_gather(b=b):
                    pltpu.async_copy(h_hbm.at[sidx.at[j0 + NBUF + b]],
                                     rows.at[b], gsem.at[b])

            return carry

        lax.fori_loop(0, NGROUP, _group, 0)
        plsc.subcore_barrier()

        pltpu.sync_copy(acc.at[pl.ds(r0, ROWS_PW)],
                        out_hbm.at[c, pl.ds(r0, ROWS_PW)])

        @pl.when(s == 0)
        def _out_tail():
            t0 = NS * ROWS_PW
            pltpu.sync_copy(acc.at[pl.ds(t0, N - t0)],
                            out_hbm.at[c, pl.ds(t0, N - t0)])

    return _prop


_prop128 = _make_prop(128, K128, False)
_prop16 = _make_prop(16, K16, False, NBUF=10)


KD = 125
NCHUNK_D = EPW // KD
NGROUP_D = NCHUNK_D // NBUF


@functools.partial(
    pl.kernel,
    out_type=jax.ShapeDtypeStruct((NC, N, 8), jnp.float32),
    mesh=_sc_mesh,
    scratch_types=[
        pltpu.VMEM_SHARED((N, 8), jnp.float32),
        pltpu.VMEM((NCHUNK_D, KD), jnp.int32),
        pltpu.VMEM((ROWS_PW, 8), jnp.float32),
        pltpu.SemaphoreType.DMA((NBUF,)),
    ],
    compiler_params=pltpu.CompilerParams(use_tc_tiling_on_sc=False),
)
def _deg_kernel(dst_hbm, ones_hbm, out_hbm, acc, didx, ones_v, ssem):
    c = lax.axis_index("c")
    s = lax.axis_index("s")
    wid = c * NS + s

    pltpu.sync_copy(dst_hbm.at[wid], didx)
    pltpu.sync_copy(ones_hbm, ones_v)

    r0 = pl.multiple_of(s * ROWS_PW, 8)
    pltpu.sync_copy(ones_v, acc.at[pl.ds(r0, ROWS_PW)])

    @pl.when(s == 0)
    def _init_tail():
        t0 = NS * ROWS_PW
        pltpu.sync_copy(ones_v.at[pl.ds(0, N - t0)], acc.at[pl.ds(t0, N - t0)])

    plsc.subcore_barrier()

    src_rows = ones_v.at[pl.ds(0, KD)]

    def _group(g, carry):
        j0 = g * NBUF
        descs = []
        for b in range(NBUF):
            descs.append(pltpu.async_copy(
                src_rows, acc.at[didx.at[j0 + b]], ssem.at[b], add=True))
        for b in range(NBUF):
            descs[b].wait()
        return carry

    lax.fori_loop(0, NGROUP_D, _group, 0)
    plsc.subcore_barrier()

    pltpu.sync_copy(acc.at[pl.ds(r0, ROWS_PW)], out_hbm.at[c, pl.ds(r0, ROWS_PW)])

    @pl.when(s == 0)
    def _out_tail():
        t0 = NS * ROWS_PW
        pltpu.sync_copy(acc.at[pl.ds(t0, N - t0)], out_hbm.at[c, pl.ds(t0, N - t0)])


def _tc1_body(degq_ref, x_ref, w1_ref, h1s_ref, dinv_ref):
    deg = degq_ref[0, :, 0] + degq_ref[1, :, 0] - 1.0
    dinv = lax.rsqrt(deg)
    dinv_ref[...] = dinv
    h1 = jnp.dot(x_ref[...], w1_ref[...], preferred_element_type=jnp.float32)
    h1s_ref[...] = h1 * dinv[:, None]


def _tc2_body(p_ref, h1s_ref, dinv_ref, b1_ref, w2_ref, h2s_ref):
    dinv = dinv_ref[...]
    raw = p_ref[0] + p_ref[1] - h1s_ref[...]
    out1 = raw * dinv[:, None] + b1_ref[...][None, :]
    a = jnp.maximum(out1, 0.0)
    h2 = jnp.dot(a, w2_ref[...], preferred_element_type=jnp.float32)
    h2s_ref[...] = h2 * dinv[:, None]


def _tc3_body(q_ref, h2s_ref, dinv_ref, b2_ref, out_ref):
    raw = q_ref[0] + q_ref[1] - h2s_ref[...]
    out_ref[...] = raw * dinv_ref[...][:, None] + b2_ref[...][None, :]


_tc1 = pl.pallas_call(
    _tc1_body,
    out_shape=(jax.ShapeDtypeStruct((N, 128), jnp.float32),
               jax.ShapeDtypeStruct((N,), jnp.float32)),
)

_tc2 = pl.pallas_call(
    _tc2_body,
    out_shape=jax.ShapeDtypeStruct((N, 16), jnp.float32),
)

_tc3 = pl.pallas_call(
    _tc3_body,
    out_shape=jax.ShapeDtypeStruct((N, 16), jnp.float32),
)


def kernel(x, train_pos_edge_index, W1, b1, W2, b2):
    src = train_pos_edge_index[0].astype(jnp.int32)
    dst = train_pos_edge_index[1].astype(jnp.int32)
    src16 = src.reshape(NW, EPW // K16, K16)
    dst16 = dst.reshape(NW, EPW // K16, K16)
    src128 = src.reshape(NW, EPW // K128, K128)
    dst128 = dst.reshape(NW, EPW // K128, K128)
    ones = jnp.ones((ROWS_PW, 8), jnp.float32)
    degq = _deg_kernel(dst16, ones)
    h1s, dinv = _tc1(degq, x, W1)
    p = _prop128(h1s, src128, dst128)
    h2s = _tc2(p, h1s, dinv, b1, W2)
    q = _prop16(h2s, src16, dst16)
    return _tc3(q, h2s, dinv, b2)


# speedup vs baseline: 1.1592x; 1.0003x over previous
import functools

import jax
import jax.numpy as jnp
from jax import lax
from jax.experimental import pallas as pl
from jax.experimental.pallas import tpu as pltpu
from jax.experimental.pallas import tpu_sc as plsc

N = 10000
E = 320000
NC = 2
NS = 16
NW = NC * NS
EPW = E // NW
ROWS_PW = 624
NBUF = 5
K128 = 40
K16 = 125

_sc_mesh = plsc.VectorSubcoreMesh(core_axis_name="c", subcore_axis_name="s",
                                  num_cores=NC, num_subcores=NS)


def _make_prop(D, K, tc_tiling, NBUF=NBUF):
    NCHUNK = EPW // K
    NGROUP = NCHUNK // NBUF
    assert NCHUNK * K == EPW and NGROUP * NBUF == NCHUNK and K <= 128

    @functools.partial(
        pl.kernel,
        out_type=jax.ShapeDtypeStruct((NC, N, D), jnp.float32),
        mesh=_sc_mesh,
        scratch_types=[
            pltpu.VMEM_SHARED((N, D), jnp.float32),
            pltpu.VMEM((NCHUNK, K), jnp.int32),
            pltpu.VMEM((NCHUNK, K), jnp.int32),
            pltpu.VMEM((NBUF, K, D), jnp.float32),
            pltpu.SemaphoreType.DMA((NBUF,)),
            pltpu.SemaphoreType.DMA((NBUF,)),
        ],
        compiler_params=pltpu.CompilerParams(use_tc_tiling_on_sc=tc_tiling),
    )
    def _prop(h_hbm, src_hbm, dst_hbm, out_hbm, acc, sidx, didx, rows,
              gsem, ssem):
        c = lax.axis_index("c")
        s = lax.axis_index("s")
        wid = c * NS + s

        pltpu.sync_copy(src_hbm.at[wid], sidx)
        pltpu.sync_copy(dst_hbm.at[wid], didx)

        r0 = pl.multiple_of(s * ROWS_PW, 8)
        pltpu.sync_copy(h_hbm.at[pl.ds(r0, ROWS_PW)], acc.at[pl.ds(r0, ROWS_PW)])

        @pl.when(s == 0)
        def _init_tail():
            t0 = NS * ROWS_PW
            pltpu.sync_copy(h_hbm.at[pl.ds(t0, N - t0)],
                            acc.at[pl.ds(t0, N - t0)])

        plsc.subcore_barrier()

        for b in range(NBUF):
            pltpu.async_copy(h_hbm.at[sidx.at[b]], rows.at[b], gsem.at[b])

        def _group(g, carry):
            j0 = g * NBUF
            descs = []
            for b in range(NBUF):
                pltpu.make_async_copy(h_hbm.at[sidx.at[b]], rows.at[b],
                                      gsem.at[b]).wait()
                descs.append(pltpu.async_copy(
                    rows.at[b], acc.at[didx.at[j0 + b]], ssem.at[b],
                    add=True))
            for b in range(NBUF):
                descs[b].wait()

                @pl.when(g < NGROUP - 1)
                def _next_gather(b=b):
                    pltpu.async_copy(h_hbm.at[sidx.at[j0 + NBUF + b]],
                                     rows.at[b], gsem.at[b])

            return carry

        lax.fori_loop(0, NGROUP, _group, 0)
        plsc.subcore_barrier()

        pltpu.sync_copy(acc.at[pl.ds(r0, ROWS_PW)],
                        out_hbm.at[c, pl.ds(r0, ROWS_PW)])

        @pl.when(s == 0)
        def _out_tail():
            t0 = NS * ROWS_PW
            pltpu.sync_copy(acc.at[pl.ds(t0, N - t0)],
                            out_hbm.at[c, pl.ds(t0, N - t0)])

    return _prop


_prop128 = _make_prop(128, K128, False)
_prop16 = _make_prop(16, K16, False, NBUF=8)


KD = 125
NCHUNK_D = EPW // KD
NGROUP_D = NCHUNK_D // NBUF


@functools.partial(
    pl.kernel,
    out_type=jax.ShapeDtypeStruct((NC, N, 8), jnp.float32),
    mesh=_sc_mesh,
    scratch_types=[
        pltpu.VMEM_SHARED((N, 8), jnp.float32),
        pltpu.VMEM((NCHUNK_D, KD), jnp.int32),
        pltpu.VMEM((ROWS_PW, 8), jnp.float32),
        pltpu.SemaphoreType.DMA((NBUF,)),
    ],
    compiler_params=pltpu.CompilerParams(use_tc_tiling_on_sc=False),
)
def _deg_kernel(dst_hbm, ones_hbm, out_hbm, acc, didx, ones_v, ssem):
    c = lax.axis_index("c")
    s = lax.axis_index("s")
    wid = c * NS + s

    pltpu.sync_copy(dst_hbm.at[wid], didx)
    pltpu.sync_copy(ones_hbm, ones_v)

    r0 = pl.multiple_of(s * ROWS_PW, 8)
    pltpu.sync_copy(ones_v, acc.at[pl.ds(r0, ROWS_PW)])

    @pl.when(s == 0)
    def _init_tail():
        t0 = NS * ROWS_PW
        pltpu.sync_copy(ones_v.at[pl.ds(0, N - t0)], acc.at[pl.ds(t0, N - t0)])

    plsc.subcore_barrier()

    src_rows = ones_v.at[pl.ds(0, KD)]

    def _group(g, carry):
        j0 = g * NBUF
        descs = []
        for b in range(NBUF):
            descs.append(pltpu.async_copy(
                src_rows, acc.at[didx.at[j0 + b]], ssem.at[b], add=True))
        for b in range(NBUF):
            descs[b].wait()
        return carry

    lax.fori_loop(0, NGROUP_D, _group, 0)
    plsc.subcore_barrier()

    pltpu.sync_copy(acc.at[pl.ds(r0, ROWS_PW)], out_hbm.at[c, pl.ds(r0, ROWS_PW)])

    @pl.when(s == 0)
    def _out_tail():
        t0 = NS * ROWS_PW
        pltpu.sync_copy(acc.at[pl.ds(t0, N - t0)], out_hbm.at[c, pl.ds(t0, N - t0)])


def _tc1_body(degq_ref, x_ref, w1_ref, h1s_ref, dinv_ref):
    deg = degq_ref[0, :, 0] + degq_ref[1, :, 0] - 1.0
    dinv = lax.rsqrt(deg)
    dinv_ref[...] = dinv
    h1 = jnp.dot(x_ref[...], w1_ref[...], preferred_element_type=jnp.float32)
    h1s_ref[...] = h1 * dinv[:, None]


def _tc2_body(p_ref, h1s_ref, dinv_ref, b1_ref, w2_ref, h2s_ref):
    dinv = dinv_ref[...]
    raw = p_ref[0] + p_ref[1] - h1s_ref[...]
    out1 = raw * dinv[:, None] + b1_ref[...][None, :]
    a = jnp.maximum(out1, 0.0)
    h2 = jnp.dot(a, w2_ref[...], preferred_element_type=jnp.float32)
    h2s_ref[...] = h2 * dinv[:, None]


def _tc3_body(q_ref, h2s_ref, dinv_ref, b2_ref, out_ref):
    raw = q_ref[0] + q_ref[1] - h2s_ref[...]
    out_ref[...] = raw * dinv_ref[...][:, None] + b2_ref[...][None, :]


_tc1 = pl.pallas_call(
    _tc1_body,
    out_shape=(jax.ShapeDtypeStruct((N, 128), jnp.float32),
               jax.ShapeDtypeStruct((N,), jnp.float32)),
)

_tc2 = pl.pallas_call(
    _tc2_body,
    out_shape=jax.ShapeDtypeStruct((N, 16), jnp.float32),
)

_tc3 = pl.pallas_call(
    _tc3_body,
    out_shape=jax.ShapeDtypeStruct((N, 16), jnp.float32),
)


def kernel(x, train_pos_edge_index, W1, b1, W2, b2):
    src = train_pos_edge_index[0].astype(jnp.int32)
    dst = train_pos_edge_index[1].astype(jnp.int32)
    src16 = src.reshape(NW, EPW // K16, K16)
    dst16 = dst.reshape(NW, EPW // K16, K16)
    src128 = src.reshape(NW, EPW // K128, K128)
    dst128 = dst.reshape(NW, EPW // K128, K128)
    ones = jnp.ones((ROWS_PW, 8), jnp.float32)
    degq = _deg_kernel(dst16, ones)
    h1s, dinv = _tc1(degq, x, W1)
    p = _prop128(h1s, src128, dst128)
    h2s = _tc2(p, h1s, dinv, b1, W2)
    q = _prop16(h2s, src16, dst16)
    return _tc3(q, h2s, dinv, b2)
